# baseline (speedup 1.0000x reference)
import functools

import jax
import jax.numpy as jnp
from jax import lax
from jax.experimental import pallas as pl
from jax.experimental.pallas import tpu as pltpu
from jax.experimental.pallas import tpu_sc as plsc

_NJ = 17
_PAIRS_LOCAL = (((0, 1), (3, 4)), ((1, 2), (4, 5)))
_EPS = 1e-5
_NBUF = 3
_TL = 128


def _rsqrt(s):
    i = plsc.bitcast(s, jnp.int32)
    y = plsc.bitcast(jnp.int32(0x5F3759DF) - (i >> 1), jnp.float32)
    y = y * (1.5 - 0.5 * s * y * y)
    return y


@functools.cache
def _make_kernel(n):
    info = plsc.get_sparse_core_info()
    nw = info.num_cores * info.num_subcores
    lanes = info.num_lanes
    nt = n // _TL
    tpw = nt // nw
    ct = min(4, tpw)
    nchunks = tpw // ct
    ngroups = ct * (_TL // lanes)
    gpt = _TL // lanes
    mesh = plsc.VectorSubcoreMesh(core_axis_name="c", subcore_axis_name="s")

    blk = jax.ShapeDtypeStruct((2, nt, 4, _TL), jnp.float32)

    @functools.partial(
        pl.kernel,
        out_type=(blk, blk, blk, blk),
        mesh=mesh,
        scratch_types=[pltpu.VMEM((6, ct, 4, _TL), jnp.float32) for _ in range(2 * _NBUF)]
        + [pltpu.VMEM((8, ct, 4, _TL), jnp.float32) for _ in range(_NBUF)]
        + [pltpu.SemaphoreType.DMA((_NBUF,)) for _ in range(3)],
        compiler_params=pltpu.CompilerParams(
            needs_layout_passes=False, use_tc_tiling_on_sc=False
        ),
    )
    def _k(a_hbm, b_hbm, y1_hbm, y2_hbm, y3_hbm, y4_hbm, *rest):
        abufs = rest[:_NBUF]
        bbufs = rest[_NBUF:2 * _NBUF]
        obufs = rest[2 * _NBUF:3 * _NBUF]
        in_semsa, in_semsb, out_sems = rest[3 * _NBUF:]
        wid = lax.axis_index("s") * info.num_cores + lax.axis_index("c")
        base0 = wid * tpw

        def start_in(ci):
            b = ci % _NBUF
            ds = pl.ds(base0 + ci * ct, ct)
            return (
                pltpu.async_copy(a_hbm.at[:, ds], abufs[b], in_semsa.at[b]),
                pltpu.async_copy(b_hbm.at[:, ds], bbufs[b], in_semsb.at[b]),
            )

        def start_out(ci):
            b = ci % _NBUF
            ds = pl.ds(base0 + ci * ct, ct)
            ob = obufs[b]
            return (
                pltpu.async_copy(ob.at[pl.ds(0, 2)], y1_hbm.at[:, ds], out_sems.at[b]),
                pltpu.async_copy(ob.at[pl.ds(2, 2)], y2_hbm.at[:, ds], out_sems.at[b]),
                pltpu.async_copy(ob.at[pl.ds(4, 2)], y3_hbm.at[:, ds], out_sems.at[b]),
                pltpu.async_copy(ob.at[pl.ds(6, 2)], y4_hbm.at[:, ds], out_sems.at[b]),
            )

        def wait(handles):
            for h in handles:
                h.wait()

        def compute(ci):
            b = ci % _NBUF
            obuf = obufs[b]

            def do_group(g, c2):
                tt = g // gpt
                sl = pl.ds((g % gpt) * lanes, lanes)
                for buf, off in ((abufs[b], 0), (bbufs[b], 4)):
                    jd = {j: [buf[j, tt, k, sl] for k in range(3)] for j in range(6)}

                    for (ls, le), (rs, re) in _PAIRS_LOCAL:
                        bl = [jd[le][k] - jd[ls][k] for k in range(3)]
                        br = [jd[re][k] - jd[rs][k] for k in range(3)]
                        s2l = bl[0] * bl[0] + bl[1] * bl[1] + bl[2] * bl[2]
                        s2r = br[0] * br[0] + br[1] * br[1] + br[2] * br[2]
                        yl = _rsqrt(s2l)
                        yr = _rsqrt(s2r)
                        lenl = s2l * yl
                        lenr = s2r * yr
                        invl = 1.0 / (lenl + _EPS)
                        invr = 1.0 / (lenr + _EPS)
                        hd = 0.5 * (lenl - lenr)
                        tl = hd * invl
                        tr = hd * invr
                        jd[le] = [jd[le][k] - tl * bl[k] for k in range(3)]
                        jd[re] = [jd[re][k] + tr * br[k] for k in range(3)]

                    for row, j in enumerate((1, 2, 4, 5)):
                        for k in range(3):
                            obuf[off + row, tt, k, sl] = jd[j][k]
                return c2

            lax.fori_loop(0, ngroups, do_group, 0)

        h_in, h_out = {}, {}
        for k in range(min(_NBUF - 1, nchunks)):
            h_in[k] = start_in(k)
        for ci in range(nchunks):
            wait(h_in[ci])
            compute(ci)
            h_out[ci] = start_out(ci)
            nxt = ci + _NBUF - 1
            if nxt < nchunks:
                if ci >= 1:
                    wait(h_out[ci - 1])
                h_in[nxt] = start_in(nxt)
        for ci in range(max(0, nchunks - _NBUF), nchunks):
            if ci in h_out:
                wait(h_out[ci])

    return _k


def kernel(joints_3d):
    n = joints_3d.shape[0]
    nt = n // _TL
    nslab = 2
    nts = nt // nslab
    jt = jnp.transpose(joints_3d, (1, 2, 0))
    jt4 = jt.reshape(_NJ, 3, nt, _TL).transpose(0, 2, 1, 3)
    parts = []
    for s in range(nslab):
        sl = slice(s * nts, (s + 1) * nts)
        jt4p = jnp.pad(jt4[:, sl], ((0, 0), (0, 0), (0, 1), (0, 0)))
        parts.append(_make_kernel(nts * _TL)(jt4p[1:7], jt4p[11:17]))
    y1, y2, y3, y4 = (jnp.concatenate([p[i] for p in parts], axis=1)
                      for i in range(4))
    out_jt4 = jnp.concatenate(
        [jt4[0:2], y1[:, :, 0:3, :], jt4[4:5], y2[:, :, 0:3, :],
         jt4[7:12], y3[:, :, 0:3, :], jt4[14:15], y4[:, :, 0:3, :]], axis=0)
    out_jt = out_jt4.transpose(0, 2, 1, 3).reshape(_NJ, 3, n)
    return jnp.transpose(out_jt, (2, 0, 1))


# speedup vs baseline: 1.6175x; 1.6175x over previous
import functools

import jax
import jax.numpy as jnp
from jax import lax
from jax.experimental import pallas as pl
from jax.experimental.pallas import tpu as pltpu
from jax.experimental.pallas import tpu_sc as plsc

_NJ = 17
_PAIRS_LOCAL = (((0, 1), (3, 4)), ((1, 2), (4, 5)))
_EPS = 1e-5
_NBUF = 3
_TL = 128


def _rsqrt(s):
    i = plsc.bitcast(s, jnp.int32)
    y = plsc.bitcast(jnp.int32(0x5F3759DF) - (i >> 1), jnp.float32)
    y = y * (1.5 - 0.5 * s * y * y)
    return y


@functools.cache
def _make_kernel(n):
    info = plsc.get_sparse_core_info()
    nw = info.num_cores * info.num_subcores
    lanes = info.num_lanes
    nt = n // _TL
    tpw = nt // nw
    ct = min(4, tpw)
    nchunks = tpw // ct
    ngroups = ct * (_TL // lanes)
    gpt = _TL // lanes
    mesh = plsc.VectorSubcoreMesh(core_axis_name="c", subcore_axis_name="s")

    blk = jax.ShapeDtypeStruct((2, nt, 4, _TL), jnp.float32)

    @functools.partial(
        pl.kernel,
        out_type=(blk, blk, blk, blk),
        mesh=mesh,
        scratch_types=[pltpu.VMEM((6, ct, 3, _TL), jnp.float32) for _ in range(2 * _NBUF)]
        + [pltpu.VMEM((8, ct, 3, _TL), jnp.float32) for _ in range(_NBUF)]
        + [pltpu.SemaphoreType.DMA((_NBUF,)) for _ in range(3)],
        compiler_params=pltpu.CompilerParams(
            needs_layout_passes=False, use_tc_tiling_on_sc=False
        ),
    )
    def _k(a_hbm, b_hbm, y1_hbm, y2_hbm, y3_hbm, y4_hbm, *rest):
        abufs = rest[:_NBUF]
        bbufs = rest[_NBUF:2 * _NBUF]
        obufs = rest[2 * _NBUF:3 * _NBUF]
        in_semsa, in_semsb, out_sems = rest[3 * _NBUF:]
        wid = lax.axis_index("s") * info.num_cores + lax.axis_index("c")
        base0 = wid * tpw

        def start_in(ci):
            b = ci % _NBUF
            ds = pl.ds(base0 + ci * ct, ct)
            c3 = pl.ds(0, 3)
            return (
                pltpu.async_copy(a_hbm.at[:, ds, c3, :], abufs[b], in_semsa.at[b]),
                pltpu.async_copy(b_hbm.at[:, ds, c3, :], bbufs[b], in_semsb.at[b]),
            )

        def start_out(ci):
            b = ci % _NBUF
            ds = pl.ds(base0 + ci * ct, ct)
            ob = obufs[b]
            c3 = pl.ds(0, 3)
            return (
                pltpu.async_copy(ob.at[pl.ds(0, 2)], y1_hbm.at[:, ds, c3, :], out_sems.at[b]),
                pltpu.async_copy(ob.at[pl.ds(2, 2)], y2_hbm.at[:, ds, c3, :], out_sems.at[b]),
                pltpu.async_copy(ob.at[pl.ds(4, 2)], y3_hbm.at[:, ds, c3, :], out_sems.at[b]),
                pltpu.async_copy(ob.at[pl.ds(6, 2)], y4_hbm.at[:, ds, c3, :], out_sems.at[b]),
            )

        def wait(handles):
            for h in handles:
                h.wait()

        def compute(ci):
            b = ci % _NBUF
            obuf = obufs[b]

            def do_group(g, c2):
                tt = g // gpt
                sl = pl.ds((g % gpt) * lanes, lanes)
                for buf, off in ((abufs[b], 0), (bbufs[b], 4)):
                    jd = {j: [buf[j, tt, k, sl] for k in range(3)] for j in range(6)}

                    for (ls, le), (rs, re) in _PAIRS_LOCAL:
                        bl = [jd[le][k] - jd[ls][k] for k in range(3)]
                        br = [jd[re][k] - jd[rs][k] for k in range(3)]
                        s2l = bl[0] * bl[0] + bl[1] * bl[1] + bl[2] * bl[2]
                        s2r = br[0] * br[0] + br[1] * br[1] + br[2] * br[2]
                        yl = _rsqrt(s2l)
                        yr = _rsqrt(s2r)
                        lenl = s2l * yl
                        lenr = s2r * yr
                        invl = 1.0 / (lenl + _EPS)
                        invr = 1.0 / (lenr + _EPS)
                        hd = 0.5 * (lenl - lenr)
                        tl = hd * invl
                        tr = hd * invr
                        jd[le] = [jd[le][k] - tl * bl[k] for k in range(3)]
                        jd[re] = [jd[re][k] + tr * br[k] for k in range(3)]

                    for row, j in enumerate((1, 2, 4, 5)):
                        for k in range(3):
                            obuf[off + row, tt, k, sl] = jd[j][k]
                return c2

            lax.fori_loop(0, ngroups, do_group, 0)

        h_in, h_out = {}, {}
        for k in range(min(_NBUF - 1, nchunks)):
            h_in[k] = start_in(k)
        for ci in range(nchunks):
            wait(h_in[ci])
            compute(ci)
            h_out[ci] = start_out(ci)
            nxt = ci + _NBUF - 1
            if nxt < nchunks:
                if ci >= 1:
                    wait(h_out[ci - 1])
                h_in[nxt] = start_in(nxt)
        for ci in range(max(0, nchunks - _NBUF), nchunks):
            if ci in h_out:
                wait(h_out[ci])

    return _k


def kernel(joints_3d):
    n = joints_3d.shape[0]
    nt = n // _TL
    jt = jnp.transpose(joints_3d, (1, 2, 0))
    jt4 = jt.reshape(_NJ, 3, nt, _TL).transpose(0, 2, 1, 3)
    jt4p = jnp.pad(jt4, ((0, 0), (0, 0), (0, 1), (0, 0)))
    y1, y2, y3, y4 = _make_kernel(n)(jt4p[1:7], jt4p[11:17])
    out_jt4 = jnp.concatenate(
        [jt4[0:2], y1[:, :, 0:3, :], jt4[4:5], y2[:, :, 0:3, :],
         jt4[7:12], y3[:, :, 0:3, :], jt4[14:15], y4[:, :, 0:3, :]], axis=0)
    out_jt = out_jt4.transpose(0, 2, 1, 3).reshape(_NJ, 3, n)
    return jnp.transpose(out_jt, (2, 0, 1))


# speedup vs baseline: 1.6329x; 1.0096x over previous
import functools

import jax
import jax.numpy as jnp
from jax import lax
from jax.experimental import pallas as pl
from jax.experimental.pallas import tpu as pltpu
from jax.experimental.pallas import tpu_sc as plsc

_NJ = 17
_PAIRS_LOCAL = (((0, 1), (3, 4)), ((1, 2), (4, 5)))
_EPS = 1e-5
_NBUF = 4
_TL = 128


def _rsqrt(s):
    i = plsc.bitcast(s, jnp.int32)
    y = plsc.bitcast(jnp.int32(0x5F3759DF) - (i >> 1), jnp.float32)
    y = y * (1.5 - 0.5 * s * y * y)
    return y


@functools.cache
def _make_kernel(n):
    info = plsc.get_sparse_core_info()
    nw = info.num_cores * info.num_subcores
    lanes = info.num_lanes
    nt = n // _TL
    tpw = nt // nw
    ct = min(4, tpw)
    nchunks = tpw // ct
    ngroups = ct * (_TL // lanes)
    gpt = _TL // lanes
    mesh = plsc.VectorSubcoreMesh(core_axis_name="c", subcore_axis_name="s")

    blk = jax.ShapeDtypeStruct((2, nt, 4, _TL), jnp.float32)

    @functools.partial(
        pl.kernel,
        out_type=(blk, blk, blk, blk),
        mesh=mesh,
        scratch_types=[pltpu.VMEM((6, ct, 3, _TL), jnp.float32) for _ in range(2 * _NBUF)]
        + [pltpu.VMEM((8, ct, 3, _TL), jnp.float32) for _ in range(_NBUF)]
        + [pltpu.SemaphoreType.DMA((_NBUF,)) for _ in range(3)],
        compiler_params=pltpu.CompilerParams(
            needs_layout_passes=False, use_tc_tiling_on_sc=False
        ),
    )
    def _k(a_hbm, b_hbm, y1_hbm, y2_hbm, y3_hbm, y4_hbm, *rest):
        abufs = rest[:_NBUF]
        bbufs = rest[_NBUF:2 * _NBUF]
        obufs = rest[2 * _NBUF:3 * _NBUF]
        in_semsa, in_semsb, out_sems = rest[3 * _NBUF:]
        wid = lax.axis_index("s") * info.num_cores + lax.axis_index("c")
        base0 = wid * tpw

        def start_in(ci):
            b = ci % _NBUF
            ds = pl.ds(base0 + ci * ct, ct)
            c3 = pl.ds(0, 3)
            return (
                pltpu.async_copy(a_hbm.at[:, ds, c3, :], abufs[b], in_semsa.at[b]),
                pltpu.async_copy(b_hbm.at[:, ds, c3, :], bbufs[b], in_semsb.at[b]),
            )

        def start_out(ci):
            b = ci % _NBUF
            ds = pl.ds(base0 + ci * ct, ct)
            ob = obufs[b]
            c3 = pl.ds(0, 3)
            return (
                pltpu.async_copy(ob.at[pl.ds(0, 2)], y1_hbm.at[:, ds, c3, :], out_sems.at[b]),
                pltpu.async_copy(ob.at[pl.ds(2, 2)], y2_hbm.at[:, ds, c3, :], out_sems.at[b]),
                pltpu.async_copy(ob.at[pl.ds(4, 2)], y3_hbm.at[:, ds, c3, :], out_sems.at[b]),
                pltpu.async_copy(ob.at[pl.ds(6, 2)], y4_hbm.at[:, ds, c3, :], out_sems.at[b]),
            )

        def wait(handles):
            for h in handles:
                h.wait()

        def compute(ci):
            b = ci % _NBUF
            obuf = obufs[b]

            def do_group(g, c2):
                tt = g // gpt
                sl = pl.ds((g % gpt) * lanes, lanes)
                for buf, off in ((abufs[b], 0), (bbufs[b], 4)):
                    jd = {j: [buf[j, tt, k, sl] for k in range(3)] for j in range(6)}

                    for (ls, le), (rs, re) in _PAIRS_LOCAL:
                        bl = [jd[le][k] - jd[ls][k] for k in range(3)]
                        br = [jd[re][k] - jd[rs][k] for k in range(3)]
                        s2l = bl[0] * bl[0] + bl[1] * bl[1] + bl[2] * bl[2]
                        s2r = br[0] * br[0] + br[1] * br[1] + br[2] * br[2]
                        yl = _rsqrt(s2l)
                        yr = _rsqrt(s2r)
                        lenl = s2l * yl
                        lenr = s2r * yr
                        invl = 1.0 / (lenl + _EPS)
                        invr = 1.0 / (lenr + _EPS)
                        hd = 0.5 * (lenl - lenr)
                        tl = hd * invl
                        tr = hd * invr
                        jd[le] = [jd[le][k] - tl * bl[k] for k in range(3)]
                        jd[re] = [jd[re][k] + tr * br[k] for k in range(3)]

                    for row, j in enumerate((1, 2, 4, 5)):
                        for k in range(3):
                            obuf[off + row, tt, k, sl] = jd[j][k]
                return c2

            lax.fori_loop(0, ngroups, do_group, 0)

        h_in, h_out = {}, {}
        for k in range(min(_NBUF - 1, nchunks)):
            h_in[k] = start_in(k)
        for ci in range(nchunks):
            wait(h_in[ci])
            compute(ci)
            h_out[ci] = start_out(ci)
            nxt = ci + _NBUF - 1
            if nxt < nchunks:
                if ci >= 1:
                    wait(h_out[ci - 1])
                h_in[nxt] = start_in(nxt)
        for ci in range(max(0, nchunks - _NBUF), nchunks):
            if ci in h_out:
                wait(h_out[ci])

    return _k


def kernel(joints_3d):
    n = joints_3d.shape[0]
    nt = n // _TL
    jt = jnp.transpose(joints_3d, (1, 2, 0))
    jt4 = jt.reshape(_NJ, 3, nt, _TL).transpose(0, 2, 1, 3)
    jt4p = jnp.pad(jt4, ((0, 0), (0, 0), (0, 1), (0, 0)))
    y1, y2, y3, y4 = _make_kernel(n)(jt4p[1:7], jt4p[11:17])
    out_jt4 = jnp.concatenate(
        [jt4[0:2], y1[:, :, 0:3, :], jt4[4:5], y2[:, :, 0:3, :],
         jt4[7:12], y3[:, :, 0:3, :], jt4[14:15], y4[:, :, 0:3, :]], axis=0)
    out_jt = out_jt4.transpose(0, 2, 1, 3).reshape(_NJ, 3, n)
    return jnp.transpose(out_jt, (2, 0, 1))
